# Initial kernel scaffold; baseline (speedup 1.0000x reference)
import jax
import jax.numpy as jnp
from jax.experimental import pallas as pl


def kernel(x, wide_w, emb, W1, b1, W2, b2, W3, b3):
    raise NotImplementedError("write your pallas kernel here")


# speedup vs baseline: 1.4863x; 1.4863x over previous
import functools

import jax
import jax.numpy as jnp
from jax import lax
from jax.experimental import pallas as pl
from jax.experimental.pallas import tpu as pltpu
from jax.experimental.pallas import tpu_sc as plsc

NUM_FIELDS = 26
FIELD_DIM = 100000
EMBED_DIM = 16
BATCH = 16384

NC, NS = 2, 16
NW = NC * NS
BF = BATCH * NUM_FIELDS
CHUNK = 128
CPT = BF // NW // CHUNK
ROWS_PER_TILE = BF // NW
K = 4
GROUPS = CPT // K


def _sc_gather(flat_idx, x_rows, emb2d, wide2d):
    mesh = plsc.VectorSubcoreMesh(
        core_axis_name="c", subcore_axis_name="s",
        num_cores=NC, num_subcores=NS)

    def body(idx_hbm, xw_hbm, emb_hbm, wide_hbm, deep_out, wide_out,
             idx_v, xw_v, deep_b, wide_b, sem):
        wid = lax.axis_index("s") * NC + lax.axis_index("c")
        row0 = wid * CPT
        base = wid * ROWS_PER_TILE
        pltpu.sync_copy(idx_hbm.at[pl.ds(row0, CPT)], idx_v)
        pltpu.sync_copy(xw_hbm.at[pl.ds(row0, CPT)], xw_v)

        def chunk(j, c):
            pltpu.make_async_copy(emb_hbm.at[idx_v.at[j]], deep_b, sem).start()
            pltpu.make_async_copy(wide_hbm.at[xw_v.at[j]], wide_b, sem).start()
            pltpu.make_async_copy(emb_hbm.at[idx_v.at[j]], deep_b, sem).wait()
            pltpu.make_async_copy(wide_hbm.at[xw_v.at[j]], wide_b, sem).wait()
            pltpu.sync_copy(deep_b, deep_out.at[pl.ds(base + j * CHUNK, CHUNK)])
            pltpu.sync_copy(wide_b, wide_out.at[pl.ds(base + j * CHUNK, CHUNK)])
            return c

        lax.fori_loop(0, CPT, chunk, 0)

    return pl.kernel(
        body,
        out_type=[
            jax.ShapeDtypeStruct((BF, EMBED_DIM), jnp.float32),
            jax.ShapeDtypeStruct((BF, 1), jnp.float32),
        ],
        mesh=mesh,
        scratch_types=[
            pltpu.VMEM((CPT, CHUNK), jnp.int32),
            pltpu.VMEM((CPT, CHUNK), jnp.int32),
            pltpu.VMEM((CHUNK, EMBED_DIM), jnp.float32),
            pltpu.VMEM((CHUNK, 1), jnp.float32),
            pltpu.SemaphoreType.DMA,
        ],
        compiler_params=pltpu.CompilerParams(use_tc_tiling_on_sc=False),
    )(flat_idx, x_rows, emb2d, wide2d)


def _tc_mlp(gath, widev, W1, b1, W2, b2, W3t, b3):
    BR = 1024
    grid = BATCH // BR

    def body(x_ref, wv_ref, w1_ref, b1_ref, w2_ref, b2_ref, w3t_ref, b3_ref,
             o_ref):
        x = x_ref[...]
        h = jnp.dot(x, w1_ref[...], preferred_element_type=jnp.float32)
        h = jnp.maximum(h + b1_ref[...], 0.0)
        h = jnp.dot(h, w2_ref[...], preferred_element_type=jnp.float32)
        h = jnp.maximum(h + b2_ref[...], 0.0)
        deep = jnp.sum(h * w3t_ref[...], axis=1, keepdims=True)
        wide = jnp.sum(wv_ref[...], axis=1, keepdims=True)
        o_ref[...] = deep + wide + b3_ref[...]

    return pl.pallas_call(
        body,
        grid=(grid,),
        in_specs=[
            pl.BlockSpec((BR, NUM_FIELDS * EMBED_DIM), lambda i: (i, 0)),
            pl.BlockSpec((BR, NUM_FIELDS), lambda i: (i, 0)),
            pl.BlockSpec((NUM_FIELDS * EMBED_DIM, 64), lambda i: (0, 0)),
            pl.BlockSpec((1, 64), lambda i: (0, 0)),
            pl.BlockSpec((64, 32), lambda i: (0, 0)),
            pl.BlockSpec((1, 32), lambda i: (0, 0)),
            pl.BlockSpec((1, 32), lambda i: (0, 0)),
            pl.BlockSpec((1, 1), lambda i: (0, 0)),
        ],
        out_specs=pl.BlockSpec((BR, 1), lambda i: (i, 0)),
        out_shape=jax.ShapeDtypeStruct((BATCH, 1), jnp.float32),
    )(gath, widev, W1, b1, W2, b2, W3t, b3)


def kernel(x, wide_w, emb, W1, b1, W2, b2, W3, b3):
    offs = (jnp.arange(NUM_FIELDS, dtype=jnp.int32) * FIELD_DIM)[None, :]
    flat_idx = (x + offs).reshape(BF // CHUNK, CHUNK)
    x_rows = x.reshape(BF // CHUNK, CHUNK)
    emb2d = emb.reshape(NUM_FIELDS * FIELD_DIM, EMBED_DIM)

    deep_rows, wide_vals = _sc_gather(flat_idx, x_rows, emb2d, wide_w)

    gath = deep_rows.reshape(BATCH, NUM_FIELDS * EMBED_DIM)
    widev = wide_vals.reshape(BATCH, NUM_FIELDS)
    out = _tc_mlp(gath, widev, W1, b1.reshape(1, 64), W2, b2.reshape(1, 32),
                  W3.reshape(1, 32), b3.reshape(1, 1))
    return out.reshape(BATCH)
